# Initial kernel scaffold; baseline (speedup 1.0000x reference)
import jax
import jax.numpy as jnp
from jax.experimental import pallas as pl


def kernel(coord, numbers, charge, nbmat, afv, comb_a, comb_q, params):
    raise NotImplementedError("write your pallas kernel here")


# speedup vs baseline: 24.4112x; 24.4112x over previous
import functools

import jax
import jax.numpy as jnp
from jax import lax
from jax.experimental import pallas as pl
from jax.experimental.pallas import tpu as pltpu
from jax.experimental.pallas import tpu_sc as plsc

B, N, NB = 4, 1024, 16
NF, NS, NC = 16, 16, 8
NFT = NF * NS
RC = 5.0
ETA = 4.0
AIM_D = 256
NA = B * N
P = NA * NB

_NCORES = 2
_NSUB = 16
_NW = _NCORES * _NSUB
_APW = NA // _NW
_CH = 8
_NCHUNK = _APW // _CH
_PPW = P // _NW


@functools.lru_cache(maxsize=None)
def _get_mesh():
    return plsc.VectorSubcoreMesh(core_axis_name="c", subcore_axis_name="s")


def _wid():
    return lax.axis_index("s") * _NCORES + lax.axis_index("c")


@functools.lru_cache(maxsize=None)
def _make_sc_setup_gather():
    def body(cx_hbm, cy_hbm, cz_hbm, idxp_hbm, afv_hbm, numbers_hbm,
             cjx_hbm, cjy_hbm, cjz_hbm, a0_hbm,
             cx_v, cy_v, cz_v, idx_v, ox_v, oy_v, oz_v,
             idxa_v, rowsa_v, sema):
        wid = _wid()
        pair_base = wid * _PPW
        atom_base = wid * _APW

        pltpu.sync_copy(numbers_hbm.at[pl.ds(atom_base, _APW)], idxa_v)
        cpa = pltpu.async_copy(afv_hbm.at[idxa_v], rowsa_v, sema)

        pltpu.sync_copy(cx_hbm, cx_v)
        pltpu.sync_copy(cy_hbm, cy_v)
        pltpu.sync_copy(cz_hbm, cz_v)
        pltpu.sync_copy(idxp_hbm.at[pl.ds(pair_base, _PPW)], idx_v)

        def vec(i, _):
            iv = idx_v[pl.ds(i * 16, 16)]
            ox_v[pl.ds(i * 16, 16)] = plsc.load_gather(cx_v, [iv])
            oy_v[pl.ds(i * 16, 16)] = plsc.load_gather(cy_v, [iv])
            oz_v[pl.ds(i * 16, 16)] = plsc.load_gather(cz_v, [iv])
            return 0

        lax.fori_loop(0, _PPW // 16, vec, 0)
        pltpu.sync_copy(ox_v, cjx_hbm.at[pl.ds(pair_base, _PPW)])
        pltpu.sync_copy(oy_v, cjy_hbm.at[pl.ds(pair_base, _PPW)])
        pltpu.sync_copy(oz_v, cjz_hbm.at[pl.ds(pair_base, _PPW)])
        cpa.wait()
        pltpu.sync_copy(rowsa_v, a0_hbm.at[pl.ds(atom_base, _APW)])

    return pl.kernel(
        body,
        mesh=_get_mesh(),
        out_type=(
            jax.ShapeDtypeStruct((P,), jnp.float32),
            jax.ShapeDtypeStruct((P,), jnp.float32),
            jax.ShapeDtypeStruct((P,), jnp.float32),
            jax.ShapeDtypeStruct((NA, NFT), jnp.float32),
        ),
        scratch_types=[
            pltpu.VMEM((NA,), jnp.float32),
            pltpu.VMEM((NA,), jnp.float32),
            pltpu.VMEM((NA,), jnp.float32),
            pltpu.VMEM((_PPW,), jnp.int32),
            pltpu.VMEM((_PPW,), jnp.float32),
            pltpu.VMEM((_PPW,), jnp.float32),
            pltpu.VMEM((_PPW,), jnp.float32),
            pltpu.VMEM((_APW,), jnp.int32),
            pltpu.VMEM((_APW, NFT), jnp.float32),
            pltpu.SemaphoreType.DMA,
        ],
        compiler_params=pltpu.CompilerParams(needs_layout_passes=False),
    )


@functools.lru_cache(maxsize=None)
def _make_sc_conv(with_q: bool):
    outs = [jax.ShapeDtypeStruct((NA, 4 * NFT), jnp.float32)]
    scratch = [
        pltpu.VMEM((128,), jnp.int32),
        pltpu.VMEM((128, NFT), jnp.float32),
        pltpu.VMEM((_CH, 4 * NS * NB), jnp.float32),
        pltpu.VMEM((_CH, 4 * NFT), jnp.float32),
        pltpu.SemaphoreType.DMA,
    ]
    if with_q:
        outs.append(jax.ShapeDtypeStruct((P,), jnp.float32))
        scratch += [
            pltpu.VMEM((NA,), jnp.float32),
            pltpu.VMEM((128,), jnp.float32),
        ]

    def body(*refs):
        if with_q:
            (a_hbm, w4_hbm, idxp_hbm, q_hbm, outr_hbm, outqj_hbm,
             idx_v, rows_v, w4_v, outr_v, sem, q_v, qj_v) = refs
        else:
            (a_hbm, w4_hbm, idxp_hbm, outr_hbm,
             idx_v, rows_v, w4_v, outr_v, sem) = refs
        wid = _wid()
        base_atom = wid * _APW
        if with_q:
            pltpu.sync_copy(q_hbm, q_v)

        def chunk(c, _):
            atom0 = base_atom + c * _CH
            pair0 = atom0 * NB
            pltpu.sync_copy(idxp_hbm.at[pl.ds(pair0, 128)], idx_v)
            cp = pltpu.async_copy(a_hbm.at[idx_v], rows_v, sem)
            pltpu.sync_copy(w4_hbm.at[pl.ds(atom0, _CH)], w4_v)
            if with_q:
                for i in range(8):
                    iv = idx_v[pl.ds(i * 16, 16)]
                    qj_v[pl.ds(i * 16, 16)] = plsc.load_gather(q_v, [iv])
                pltpu.sync_copy(qj_v, outqj_hbm.at[pl.ds(pair0, 128)])
            cp.wait()

            def atom(ai, _):
                for half in range(2):
                    acc = [[jnp.zeros((16,), jnp.float32) for _ in range(8)]
                           for _ in range(4)]
                    for j in range(NB):
                        pr = ai * NB + j
                        w = [w4_v[ai, pl.ds(j * 64 + k * 16, 16)]
                             for k in range(4)]
                        row = [rows_v[pr, pl.ds((half * 8 + f) * 16, 16)]
                               for f in range(8)]
                        for k in range(4):
                            for f in range(8):
                                acc[k][f] = acc[k][f] + row[f] * w[k]
                    for k in range(4):
                        for f in range(8):
                            outr_v[ai, pl.ds(k * NFT + (half * 8 + f) * 16, 16)] = acc[k][f]
                return 0

            lax.fori_loop(0, _CH, atom, 0)
            pltpu.sync_copy(outr_v, outr_hbm.at[pl.ds(atom0, _CH)])
            return 0

        lax.fori_loop(0, _NCHUNK, chunk, 0)

    return pl.kernel(body, mesh=_get_mesh(), out_type=tuple(outs),
                     scratch_types=scratch,
                     compiler_params=pltpu.CompilerParams(
                         needs_layout_passes=False))


def _tc_pair_weights(cjx, cjy, cjz, coordpad):
    A_BLK = 256
    G = NA // A_BLK

    def body(cjx_ref, cjy_ref, cjz_ref, ci_ref, out_ref):
        d2 = jnp.zeros((A_BLK, NB), jnp.float32)
        drs = []
        for dd, cref in enumerate((cjx_ref, cjy_ref, cjz_ref)):
            dr = cref[...] - ci_ref[:, dd][:, None]
            drs.append(dr)
            d2 = d2 + dr * dr
        d = jnp.sqrt(d2 + 1e-12)
        fc = jnp.where(d < RC, 0.5 * jnp.cos(jnp.pi * d / RC) + 0.5, 0.0)
        dinv = 1.0 / (d + 1e-12)
        step = (RC - 0.8) / (NS - 1)
        shifts = 0.8 + step * lax.broadcasted_iota(
            jnp.int32, (1, 1, NS), 2).astype(jnp.float32)
        dd3 = d[:, :, None] - shifts
        gs = jnp.exp(-ETA * dd3 * dd3) * fc[:, :, None]
        out_ref[:, :, 0, :] = gs
        for dd in range(3):
            u = drs[dd] * dinv
            out_ref[:, :, 1 + dd, :] = gs * u[:, :, None]

    return pl.pallas_call(
        body,
        grid=(G,),
        in_specs=[
            pl.BlockSpec((A_BLK, NB), lambda i: (i, 0)),
            pl.BlockSpec((A_BLK, NB), lambda i: (i, 0)),
            pl.BlockSpec((A_BLK, NB), lambda i: (i, 0)),
            pl.BlockSpec((A_BLK, 16), lambda i: (i, 0)),
        ],
        out_specs=pl.BlockSpec((A_BLK, NB, 4, NS), lambda i: (i, 0, 0, 0)),
        out_shape=jax.ShapeDtypeStruct((NA, NB, 4, NS), jnp.float32),
    )(cjx, cjy, cjz, coordpad)


def _w_order(with_q):
    names = ['w1a', 'w1s', 'w1v']
    if with_q:
        names += ['w1q', 'w1sq', 'w1vq']
    names += ['b1', 'w2', 'b2', 'w3da', 'b3da', 'w3qf', 'b3qf']
    return names


def _tc_mlp(a, Rr, qj, w4, q_col, combExp, comb_q, wsl, last_linear, out_da):
    A_BLK = 256
    G = NA // A_BLK
    with_q = q_col is not None

    def body(*refs):
        i = 0
        a_ref = refs[i]; i += 1
        rr_ref = refs[i]; i += 1
        if with_q:
            qj_ref = refs[i]; i += 1
            w4_ref = refs[i]; i += 1
            q_ref = refs[i]; i += 1
        ce_ref = refs[i]; i += 1
        cq_ref = refs[i]; i += 1
        w_refs = {}
        for name in _w_order(with_q):
            w_refs[name] = refs[i]; i += 1
        out_refs = refs[i:]

        a_blk = a_ref[...]
        s_part = rr_ref[:, 0:NFT]
        ce = ce_ref[...]
        vsum = jnp.zeros((A_BLK, NF * NC), jnp.float32)
        for dd in range(3):
            vc = jnp.dot(rr_ref[:, (1 + dd) * NFT:(2 + dd) * NFT], ce,
                         preferred_element_type=jnp.float32)
            vsum = vsum + vc * vc
        vn = jnp.sqrt(vsum + 1e-8)

        h = (jnp.dot(a_blk, w_refs['w1a'][...], preferred_element_type=jnp.float32)
             + jnp.dot(s_part, w_refs['w1s'][...], preferred_element_type=jnp.float32)
             + jnp.dot(vn, w_refs['w1v'][...], preferred_element_type=jnp.float32)
             + w_refs['b1'][...])
        if with_q:
            qb = q_ref[...]
            qjb = qj_ref[...]
            rq = []
            for k in range(4):
                accq = jnp.zeros((A_BLK, NS), jnp.float32)
                for j in range(NB):
                    accq = accq + qjb[:, j][:, None] * w4_ref[:, j, k, :]
                rq.append(accq)
            cqm = cq_ref[...]
            vsq = jnp.zeros((A_BLK, NC), jnp.float32)
            for dd in range(3):
                vcq = jnp.dot(rq[1 + dd], cqm,
                              preferred_element_type=jnp.float32)
                vsq = vsq + vcq * vcq
            vnq = jnp.sqrt(vsq + 1e-8)
            h = (h + qb * w_refs['w1q'][...]
                 + jnp.dot(rq[0], w_refs['w1sq'][...], preferred_element_type=jnp.float32)
                 + jnp.dot(vnq, w_refs['w1vq'][...], preferred_element_type=jnp.float32))
        h = jax.nn.gelu(h)
        h = jax.nn.gelu(jnp.dot(h, w_refs['w2'][...],
                                preferred_element_type=jnp.float32)
                        + w_refs['b2'][...])
        da = jnp.dot(h, w_refs['w3da'][...], preferred_element_type=jnp.float32) + w_refs['b3da'][...]
        dqf = jnp.dot(h, w_refs['w3qf'][...], preferred_element_type=jnp.float32) + w_refs['b3qf'][...]
        if not last_linear:
            da = jax.nn.gelu(da)
            dqf = jax.nn.gelu(dqf)
        if out_da:
            dq = dqf[:, 0:1]
            df = dqf[:, 1:2]
            out_refs[0][...] = a_blk + da
            if with_q:
                out_refs[1][...] = q_ref[...] + dq
            else:
                out_refs[1][...] = dq
            out_refs[2][...] = df * df
        else:
            out_refs[0][...] = da

    in_arrays = [a, Rr]
    in_specs = [
        pl.BlockSpec((A_BLK, NFT), lambda i: (i, 0)),
        pl.BlockSpec((A_BLK, 4 * NFT), lambda i: (i, 0)),
    ]
    if with_q:
        in_arrays += [qj, w4, q_col]
        in_specs += [
            pl.BlockSpec((A_BLK, NB), lambda i: (i, 0)),
            pl.BlockSpec((A_BLK, NB, 4, NS), lambda i: (i, 0, 0, 0)),
            pl.BlockSpec((A_BLK, 1), lambda i: (i, 0)),
        ]
    in_arrays += [combExp, comb_q]
    in_specs += [
        pl.BlockSpec((NFT, NF * NC), lambda i: (0, 0)),
        pl.BlockSpec((NS, NC), lambda i: (0, 0)),
    ]
    for name in _w_order(with_q):
        arr = wsl[name]
        in_arrays.append(arr)
        in_specs.append(
            pl.BlockSpec(arr.shape, lambda i, _r=len(arr.shape): (0,) * _r))

    if out_da:
        out_shape = [
            jax.ShapeDtypeStruct((NA, NFT), jnp.float32),
            jax.ShapeDtypeStruct((NA, 1), jnp.float32),
            jax.ShapeDtypeStruct((NA, 1), jnp.float32),
        ]
        out_specs = [
            pl.BlockSpec((A_BLK, NFT), lambda i: (i, 0)),
            pl.BlockSpec((A_BLK, 1), lambda i: (i, 0)),
            pl.BlockSpec((A_BLK, 1), lambda i: (i, 0)),
        ]
    else:
        out_shape = [jax.ShapeDtypeStruct((NA, AIM_D), jnp.float32)]
        out_specs = [pl.BlockSpec((A_BLK, AIM_D), lambda i: (i, 0))]

    return pl.pallas_call(
        body,
        grid=(G,),
        in_specs=in_specs,
        out_specs=out_specs,
        out_shape=out_shape,
    )(*in_arrays)


def _tc_nse(qpre, f, charge):
    def body(q_ref, f_ref, c_ref, out_ref):
        qm = q_ref[...]
        fm = f_ref[...]
        dq = c_ref[...] - jnp.sum(qm, axis=1, keepdims=True)
        fs = jnp.sum(fm, axis=1, keepdims=True) + 1e-6
        out_ref[...] = qm + fm * (dq / fs)

    return pl.pallas_call(
        body,
        out_shape=jax.ShapeDtypeStruct((B, N), jnp.float32),
    )(qpre, f, charge)


def _slice_weights(ws, bs, with_q):
    w1, w2, w3 = ws
    b1, b2, b3 = bs
    d = {
        'w1a': w1[0:NFT],
        'w1s': w1[NFT:2 * NFT],
        'w1v': w1[2 * NFT:2 * NFT + NF * NC],
    }
    off = 2 * NFT + NF * NC
    if with_q:
        d['w1q'] = w1[off:off + 1]
        d['w1sq'] = w1[off + 1:off + 1 + NS]
        d['w1vq'] = w1[off + 1 + NS:off + 1 + NS + NC]
    d['b1'] = b1[None, :]
    d['w2'] = w2
    d['b2'] = b2[None, :]
    if w3.shape[1] > AIM_D:
        d['w3qf'] = w3[:, 0:2]
        d['b3qf'] = b3[None, 0:2]
        d['w3da'] = w3[:, 2:]
        d['b3da'] = b3[None, 2:]
    else:
        d['w3qf'] = w3[:, 0:2]
        d['b3qf'] = b3[None, 0:2]
        d['w3da'] = w3
        d['b3da'] = b3[None, :]
    return d


def kernel(coord, numbers, charge, nbmat, afv, comb_a, comb_q, params):
    coord_f = coord.reshape(NA, 3)
    coordpad = jnp.pad(coord_f, ((0, 0), (0, 13)))
    cx = coord_f[:, 0]
    cy = coord_f[:, 1]
    cz = coord_f[:, 2]
    idxp = (nbmat.astype(jnp.int32)
            + (jnp.arange(B, dtype=jnp.int32) * N)[:, None, None]).reshape(P)
    numbers_f = numbers.astype(jnp.int32).reshape(NA)

    cjx, cjy, cjz, a0 = _make_sc_setup_gather()(cx, cy, cz, idxp, afv,
                                                numbers_f)
    w4 = _tc_pair_weights(cjx.reshape(NA, NB), cjy.reshape(NA, NB),
                          cjz.reshape(NA, NB), coordpad)
    w4_flat = w4.reshape(NA, NB * 4 * NS)

    combExp = jnp.kron(jnp.eye(NF, dtype=jnp.float32), comb_a)
    charge_col = charge.reshape(B, 1)

    a = a0
    (Rr,) = _make_sc_conv(False)(a, w4_flat, idxp)
    wsl = _slice_weights(params['mlp0'][0], params['mlp0'][1], False)
    a, qpre, f = _tc_mlp(a, Rr, None, None, None, combExp, comb_q, wsl,
                         last_linear=True, out_da=True)
    q = _tc_nse(qpre.reshape(B, N), f.reshape(B, N), charge_col)

    Rr, qj = _make_sc_conv(True)(a, w4_flat, idxp, q.reshape(NA))
    wsl = _slice_weights(params['mlp1'][0], params['mlp1'][1], True)
    a, qpre, f = _tc_mlp(a, Rr, qj.reshape(NA, NB), w4, q.reshape(NA, 1),
                         combExp, comb_q, wsl, last_linear=False, out_da=True)
    q = _tc_nse(qpre.reshape(B, N), f.reshape(B, N), charge_col)

    Rr, qj = _make_sc_conv(True)(a, w4_flat, idxp, q.reshape(NA))
    wsl = _slice_weights(params['mlp2'][0], params['mlp2'][1], True)
    (aim,) = _tc_mlp(a, Rr, qj.reshape(NA, NB), w4, q.reshape(NA, 1),
                     combExp, comb_q, wsl, last_linear=False, out_da=False)

    return aim.reshape(B, N, AIM_D), q


# speedup vs baseline: 27.8580x; 1.1412x over previous
import functools

import jax
import jax.numpy as jnp
from jax import lax
from jax.experimental import pallas as pl
from jax.experimental.pallas import tpu as pltpu
from jax.experimental.pallas import tpu_sc as plsc

B, N, NB = 4, 1024, 16
NF, NS, NC = 16, 16, 8
NFT = NF * NS
RC = 5.0
ETA = 4.0
AIM_D = 256
NA = B * N
P = NA * NB

_NCORES = 2
_NSUB = 16
_NW = _NCORES * _NSUB
_APW = NA // _NW
_CH = 8
_NCHUNK = _APW // _CH
_PPW = P // _NW


@functools.lru_cache(maxsize=None)
def _get_mesh():
    return plsc.VectorSubcoreMesh(core_axis_name="c", subcore_axis_name="s")


def _wid():
    return lax.axis_index("s") * _NCORES + lax.axis_index("c")


@functools.lru_cache(maxsize=None)
def _make_sc_setup_gather():
    def body(cx_hbm, cy_hbm, cz_hbm, idxp_hbm, afv_hbm, numbers_hbm,
             drx_hbm, dry_hbm, drz_hbm, a0_hbm,
             cx_v, cy_v, cz_v, idx_v, ox_v, oy_v, oz_v,
             idxa_v, rowsa_v, sema):
        wid = _wid()
        pair_base = wid * _PPW
        atom_base = wid * _APW

        pltpu.sync_copy(numbers_hbm.at[pl.ds(atom_base, _APW)], idxa_v)
        cpa = pltpu.async_copy(afv_hbm.at[idxa_v], rowsa_v, sema)

        pltpu.sync_copy(cx_hbm, cx_v)
        pltpu.sync_copy(cy_hbm, cy_v)
        pltpu.sync_copy(cz_hbm, cz_v)
        pltpu.sync_copy(idxp_hbm.at[pl.ds(pair_base, _PPW)], idx_v)
        iota = lax.broadcasted_iota(jnp.int32, (16,), 0)

        def vec(i, _):
            iv = idx_v[pl.ds(i * 16, 16)]
            av = lax.shift_right_logical(pair_base + i * 16 + iota, 4)
            ox_v[pl.ds(i * 16, 16)] = (plsc.load_gather(cx_v, [iv])
                                       - plsc.load_gather(cx_v, [av]))
            oy_v[pl.ds(i * 16, 16)] = (plsc.load_gather(cy_v, [iv])
                                       - plsc.load_gather(cy_v, [av]))
            oz_v[pl.ds(i * 16, 16)] = (plsc.load_gather(cz_v, [iv])
                                       - plsc.load_gather(cz_v, [av]))
            return 0

        lax.fori_loop(0, _PPW // 16, vec, 0)
        pltpu.sync_copy(ox_v, drx_hbm.at[pl.ds(pair_base, _PPW)])
        pltpu.sync_copy(oy_v, dry_hbm.at[pl.ds(pair_base, _PPW)])
        pltpu.sync_copy(oz_v, drz_hbm.at[pl.ds(pair_base, _PPW)])
        cpa.wait()
        pltpu.sync_copy(rowsa_v, a0_hbm.at[pl.ds(atom_base, _APW)])

    return pl.kernel(
        body,
        mesh=_get_mesh(),
        out_type=(
            jax.ShapeDtypeStruct((P,), jnp.float32),
            jax.ShapeDtypeStruct((P,), jnp.float32),
            jax.ShapeDtypeStruct((P,), jnp.float32),
            jax.ShapeDtypeStruct((NA, NFT), jnp.float32),
        ),
        scratch_types=[
            pltpu.VMEM((NA,), jnp.float32),
            pltpu.VMEM((NA,), jnp.float32),
            pltpu.VMEM((NA,), jnp.float32),
            pltpu.VMEM((_PPW,), jnp.int32),
            pltpu.VMEM((_PPW,), jnp.float32),
            pltpu.VMEM((_PPW,), jnp.float32),
            pltpu.VMEM((_PPW,), jnp.float32),
            pltpu.VMEM((_APW,), jnp.int32),
            pltpu.VMEM((_APW, NFT), jnp.float32),
            pltpu.SemaphoreType.DMA,
        ],
        compiler_params=pltpu.CompilerParams(needs_layout_passes=False),
    )


@functools.lru_cache(maxsize=None)
def _make_sc_conv(with_q: bool):
    outs = [jax.ShapeDtypeStruct((NA, 4 * NFT), jnp.float32)]
    scratch = [
        pltpu.VMEM((128,), jnp.int32),
        pltpu.VMEM((128, NFT), jnp.float32),
        pltpu.VMEM((128, 64), jnp.float32),
        pltpu.VMEM((_CH, 4 * NFT), jnp.float32),
        pltpu.SemaphoreType.DMA,
    ]
    if with_q:
        outs.append(jax.ShapeDtypeStruct((NA, 4, NS), jnp.float32))
        scratch += [
            pltpu.VMEM((NA,), jnp.float32),
            pltpu.VMEM((144,), jnp.float32),
            pltpu.VMEM((_CH, 4, NS), jnp.float32),
        ]

    def body(*refs):
        if with_q:
            (a_hbm, w4_hbm, idxp_hbm, q_hbm, outr_hbm, outq_hbm,
             idx_v, rows_v, w4_v, outr_v, sem, q_v, qj_v, outq_v) = refs
        else:
            (a_hbm, w4_hbm, idxp_hbm, outr_hbm,
             idx_v, rows_v, w4_v, outr_v, sem) = refs
        wid = _wid()
        base_atom = wid * _APW
        if with_q:
            pltpu.sync_copy(q_hbm, q_v)

        def chunk(c, _):
            atom0 = base_atom + c * _CH
            pair0 = atom0 * NB
            pltpu.sync_copy(idxp_hbm.at[pl.ds(pair0, 128)], idx_v)
            cp = pltpu.async_copy(a_hbm.at[idx_v], rows_v, sem)
            pltpu.sync_copy(w4_hbm.at[pl.ds(pair0, 128)], w4_v)
            if with_q:
                for i in range(8):
                    iv = idx_v[pl.ds(i * 16, 16)]
                    qj_v[pl.ds(i * 16, 16)] = plsc.load_gather(q_v, [iv])
            cp.wait()

            def atom(ai, _):
                accq = [jnp.zeros((16,), jnp.float32) for _ in range(4)]
                for half in range(2):
                    acc = [[jnp.zeros((16,), jnp.float32) for _ in range(8)]
                           for _ in range(4)]
                    for j in range(NB):
                        pr = ai * NB + j
                        w = [w4_v[pr, pl.ds(k * 16, 16)] for k in range(4)]
                        row = [rows_v[pr, pl.ds((half * 8 + f) * 16, 16)]
                               for f in range(8)]
                        for k in range(4):
                            for f in range(8):
                                acc[k][f] = acc[k][f] + row[f] * w[k]
                        if with_q and half == 0:
                            qs = qj_v[pl.ds(pr, 16)][0]
                            for k in range(4):
                                accq[k] = accq[k] + qs * w[k]
                    for k in range(4):
                        for f in range(8):
                            outr_v[ai, pl.ds(k * NFT + (half * 8 + f) * 16, 16)] = acc[k][f]
                if with_q:
                    for k in range(4):
                        outq_v[ai, k, :] = accq[k]
                return 0

            lax.fori_loop(0, _CH, atom, 0)
            pltpu.sync_copy(outr_v, outr_hbm.at[pl.ds(atom0, _CH)])
            if with_q:
                pltpu.sync_copy(outq_v, outq_hbm.at[pl.ds(atom0, _CH)])
            return 0

        lax.fori_loop(0, _NCHUNK, chunk, 0)

    return pl.kernel(body, mesh=_get_mesh(), out_type=tuple(outs),
                     scratch_types=scratch,
                     compiler_params=pltpu.CompilerParams(
                         needs_layout_passes=False))


def _tc_pair_weights(drx, dry, drz):
    P_BLK = 2048
    G = P // P_BLK

    def body(dx_ref, dy_ref, dz_ref, out_ref):
        dx = dx_ref[...]
        dy = dy_ref[...]
        dz = dz_ref[...]
        d2 = dx * dx + dy * dy + dz * dz
        d = jnp.sqrt(d2 + 1e-12)
        fc = jnp.where(d < RC, 0.5 * jnp.cos(jnp.pi * d / RC) + 0.5, 0.0)
        dinv = 1.0 / (d + 1e-12)
        step = (RC - 0.8) / (NS - 1)
        shifts = 0.8 + step * lax.broadcasted_iota(
            jnp.int32, (1, NS), 1).astype(jnp.float32)
        dd2 = d - shifts
        gs = jnp.exp(-ETA * dd2 * dd2) * fc
        out_ref[...] = jnp.concatenate(
            [gs, gs * (dx * dinv), gs * (dy * dinv), gs * (dz * dinv)],
            axis=1)

    return pl.pallas_call(
        body,
        grid=(G,),
        in_specs=[
            pl.BlockSpec((P_BLK, 1), lambda i: (i, 0)),
            pl.BlockSpec((P_BLK, 1), lambda i: (i, 0)),
            pl.BlockSpec((P_BLK, 1), lambda i: (i, 0)),
        ],
        out_specs=pl.BlockSpec((P_BLK, 64), lambda i: (i, 0)),
        out_shape=jax.ShapeDtypeStruct((P, 64), jnp.float32),
    )(drx, dry, drz)


def _w_order(with_q):
    names = ['w1a', 'w1s', 'w1v']
    if with_q:
        names += ['w1q', 'w1sq', 'w1vq']
    names += ['b1', 'w2', 'b2', 'w3da', 'b3da', 'w3qf', 'b3qf']
    return names


def _tc_mlp(a, Rr, Rq, q_col, combExp, comb_q, wsl, last_linear, out_da):
    A_BLK = 256
    G = NA // A_BLK
    with_q = q_col is not None

    def body(*refs):
        i = 0
        a_ref = refs[i]; i += 1
        rr_ref = refs[i]; i += 1
        if with_q:
            rq_ref = refs[i]; i += 1
            q_ref = refs[i]; i += 1
        ce_ref = refs[i]; i += 1
        cq_ref = refs[i]; i += 1
        w_refs = {}
        for name in _w_order(with_q):
            w_refs[name] = refs[i]; i += 1
        out_refs = refs[i:]

        a_blk = a_ref[...]
        s_part = rr_ref[:, 0:NFT]
        ce = ce_ref[...]
        vsum = jnp.zeros((A_BLK, NF * NC), jnp.float32)
        for dd in range(3):
            vc = jnp.dot(rr_ref[:, (1 + dd) * NFT:(2 + dd) * NFT], ce,
                         preferred_element_type=jnp.float32)
            vsum = vsum + vc * vc
        vn = jnp.sqrt(vsum + 1e-8)

        h = (jnp.dot(a_blk, w_refs['w1a'][...], preferred_element_type=jnp.float32)
             + jnp.dot(s_part, w_refs['w1s'][...], preferred_element_type=jnp.float32)
             + jnp.dot(vn, w_refs['w1v'][...], preferred_element_type=jnp.float32)
             + w_refs['b1'][...])
        if with_q:
            qb = q_ref[...]
            cqm = cq_ref[...]
            vsq = jnp.zeros((A_BLK, NC), jnp.float32)
            for dd in range(3):
                vcq = jnp.dot(rq_ref[:, 1 + dd, :], cqm,
                              preferred_element_type=jnp.float32)
                vsq = vsq + vcq * vcq
            vnq = jnp.sqrt(vsq + 1e-8)
            h = (h + qb * w_refs['w1q'][...]
                 + jnp.dot(rq_ref[:, 0, :], w_refs['w1sq'][...], preferred_element_type=jnp.float32)
                 + jnp.dot(vnq, w_refs['w1vq'][...], preferred_element_type=jnp.float32))
        h = jax.nn.gelu(h)
        h = jax.nn.gelu(jnp.dot(h, w_refs['w2'][...],
                                preferred_element_type=jnp.float32)
                        + w_refs['b2'][...])
        da = jnp.dot(h, w_refs['w3da'][...], preferred_element_type=jnp.float32) + w_refs['b3da'][...]
        dqf = jnp.dot(h, w_refs['w3qf'][...], preferred_element_type=jnp.float32) + w_refs['b3qf'][...]
        if not last_linear:
            da = jax.nn.gelu(da)
            dqf = jax.nn.gelu(dqf)
        if out_da:
            dq = dqf[:, 0:1]
            df = dqf[:, 1:2]
            out_refs[0][...] = a_blk + da
            if with_q:
                out_refs[1][...] = q_ref[...] + dq
            else:
                out_refs[1][...] = dq
            out_refs[2][...] = df * df
        else:
            out_refs[0][...] = da

    in_arrays = [a, Rr]
    in_specs = [
        pl.BlockSpec((A_BLK, NFT), lambda i: (i, 0)),
        pl.BlockSpec((A_BLK, 4 * NFT), lambda i: (i, 0)),
    ]
    if with_q:
        in_arrays += [Rq, q_col]
        in_specs += [
            pl.BlockSpec((A_BLK, 4, NS), lambda i: (i, 0, 0)),
            pl.BlockSpec((A_BLK, 1), lambda i: (i, 0)),
        ]
    in_arrays += [combExp, comb_q]
    in_specs += [
        pl.BlockSpec((NFT, NF * NC), lambda i: (0, 0)),
        pl.BlockSpec((NS, NC), lambda i: (0, 0)),
    ]
    for name in _w_order(with_q):
        arr = wsl[name]
        in_arrays.append(arr)
        in_specs.append(
            pl.BlockSpec(arr.shape, lambda i, _r=len(arr.shape): (0,) * _r))

    if out_da:
        out_shape = [
            jax.ShapeDtypeStruct((NA, NFT), jnp.float32),
            jax.ShapeDtypeStruct((NA, 1), jnp.float32),
            jax.ShapeDtypeStruct((NA, 1), jnp.float32),
        ]
        out_specs = [
            pl.BlockSpec((A_BLK, NFT), lambda i: (i, 0)),
            pl.BlockSpec((A_BLK, 1), lambda i: (i, 0)),
            pl.BlockSpec((A_BLK, 1), lambda i: (i, 0)),
        ]
    else:
        out_shape = [jax.ShapeDtypeStruct((NA, AIM_D), jnp.float32)]
        out_specs = [pl.BlockSpec((A_BLK, AIM_D), lambda i: (i, 0))]

    return pl.pallas_call(
        body,
        grid=(G,),
        in_specs=in_specs,
        out_specs=out_specs,
        out_shape=out_shape,
    )(*in_arrays)


def _tc_nse(qpre, f, charge):
    def body(q_ref, f_ref, c_ref, out_ref):
        qm = q_ref[...]
        fm = f_ref[...]
        dq = c_ref[...] - jnp.sum(qm, axis=1, keepdims=True)
        fs = jnp.sum(fm, axis=1, keepdims=True) + 1e-6
        out_ref[...] = qm + fm * (dq / fs)

    return pl.pallas_call(
        body,
        out_shape=jax.ShapeDtypeStruct((B, N), jnp.float32),
    )(qpre, f, charge)


def _slice_weights(ws, bs, with_q):
    w1, w2, w3 = ws
    b1, b2, b3 = bs
    d = {
        'w1a': w1[0:NFT],
        'w1s': w1[NFT:2 * NFT],
        'w1v': w1[2 * NFT:2 * NFT + NF * NC],
    }
    off = 2 * NFT + NF * NC
    if with_q:
        d['w1q'] = w1[off:off + 1]
        d['w1sq'] = w1[off + 1:off + 1 + NS]
        d['w1vq'] = w1[off + 1 + NS:off + 1 + NS + NC]
    d['b1'] = b1[None, :]
    d['w2'] = w2
    d['b2'] = b2[None, :]
    if w3.shape[1] > AIM_D:
        d['w3qf'] = w3[:, 0:2]
        d['b3qf'] = b3[None, 0:2]
        d['w3da'] = w3[:, 2:]
        d['b3da'] = b3[None, 2:]
    else:
        d['w3qf'] = w3[:, 0:2]
        d['b3qf'] = b3[None, 0:2]
        d['w3da'] = w3
        d['b3da'] = b3[None, :]
    return d


def kernel(coord, numbers, charge, nbmat, afv, comb_a, comb_q, params):
    coord_f = coord.reshape(NA, 3)
    cx = coord_f[:, 0]
    cy = coord_f[:, 1]
    cz = coord_f[:, 2]
    idxp = (nbmat.astype(jnp.int32)
            + (jnp.arange(B, dtype=jnp.int32) * N)[:, None, None]).reshape(P)
    numbers_f = numbers.astype(jnp.int32).reshape(NA)

    drx, dry, drz, a0 = _make_sc_setup_gather()(cx, cy, cz, idxp, afv,
                                                numbers_f)
    w4 = _tc_pair_weights(drx.reshape(P, 1), dry.reshape(P, 1),
                          drz.reshape(P, 1))

    combExp = jnp.kron(jnp.eye(NF, dtype=jnp.float32), comb_a)
    charge_col = charge.reshape(B, 1)

    a = a0
    (Rr,) = _make_sc_conv(False)(a, w4, idxp)
    wsl = _slice_weights(params['mlp0'][0], params['mlp0'][1], False)
    a, qpre, f = _tc_mlp(a, Rr, None, None, combExp, comb_q, wsl,
                         last_linear=True, out_da=True)
    q = _tc_nse(qpre.reshape(B, N), f.reshape(B, N), charge_col)

    Rr, Rq = _make_sc_conv(True)(a, w4, idxp, q.reshape(NA))
    wsl = _slice_weights(params['mlp1'][0], params['mlp1'][1], True)
    a, qpre, f = _tc_mlp(a, Rr, Rq, q.reshape(NA, 1),
                         combExp, comb_q, wsl, last_linear=False, out_da=True)
    q = _tc_nse(qpre.reshape(B, N), f.reshape(B, N), charge_col)

    Rr, Rq = _make_sc_conv(True)(a, w4, idxp, q.reshape(NA))
    wsl = _slice_weights(params['mlp2'][0], params['mlp2'][1], True)
    (aim,) = _tc_mlp(a, Rr, Rq, q.reshape(NA, 1),
                     combExp, comb_q, wsl, last_linear=False, out_da=False)

    return aim.reshape(B, N, AIM_D), q
